# baseline (speedup 1.0000x reference)
import functools

import jax
import jax.numpy as jnp
from jax import lax
from jax.experimental import pallas as pl
from jax.experimental.pallas import tpu as pltpu
from jax.experimental.pallas import tpu_sc as plsc

N = 4096
LAG = 2
N_PAIRS = N * (N - 1) // 2

NW = 32
NBUCK = 8
BH = N // NBUCK

BT = 512
NT = N // BT


def _sc_fill_body(lflat, a_out, s_out,
                  l00, l10, l20, l01, l11, l21,
                  sa0, ss0, sa1, ss1, rsem, wsem):
    wid = lax.axis_index("s") * 2 + lax.axis_index("c")
    lbufs = ((l00, l10, l20), (l01, l11, l21))
    stages = ((sa0, ss0), (sa1, ss1))

    for b in range(NBUCK):
        sz = BH * (b + 1)
        rs = sz + 16
        nv = sz // 16
        clamp_hi = N_PAIRS - rs

        def row_pair(k2, c, b=b, sz=sz, rs=rs, nv=nv, clamp_hi=clamp_hi):
            for half in range(2):
                lb = lbufs[half]
                sta, sts = stages[half]
                k = 2 * k2 + half
                r = BH * b + wid + 32 * k
                off = (r * (r - 1)) // 2
                al = jnp.minimum(off - lax.rem(off, 8), clamp_hi)
                al = pl.multiple_of(al, 8)
                d = off - al

                for p in range(3):
                    pltpu.make_async_copy(
                        lflat.at[pl.ds(p * N_PAIRS + al, rs)],
                        lb[p].at[pl.ds(0, rs)], rsem).start()

                @pl.when(k2 >= 1)
                def _():
                    pltpu.make_async_copy(
                        sta.at[pl.ds(0, sz)],
                        a_out.at[0, pl.ds(0, sz)], wsem).wait()
                    pltpu.make_async_copy(
                        sts.at[pl.ds(0, sz)],
                        a_out.at[0, pl.ds(0, sz)], wsem).wait()

                for p in range(3):
                    pltpu.make_async_copy(
                        lflat.at[pl.ds(p * N_PAIRS + al, rs)],
                        lb[p].at[pl.ds(0, rs)], rsem).wait()

                def vec(m, c2, lb=lb, sta=sta, sts=sts, d=d):
                    e0 = jnp.exp(lb[0][pl.ds(d + m * 16, 16)])
                    e1 = jnp.exp(lb[1][pl.ds(d + m * 16, 16)])
                    e2 = jnp.exp(lb[2][pl.ds(d + m * 16, 16)])
                    inv = 1.0 / (e0 + e1 + e2)
                    sta[pl.ds(m * 16, 16)] = e0 * inv
                    sts[pl.ds(m * 16, 16)] = e1 * inv
                    return c2

                lax.fori_loop(0, nv, vec, 0)

                pltpu.make_async_copy(
                    sta.at[pl.ds(0, sz)],
                    a_out.at[r, pl.ds(0, sz)], wsem).start()
                pltpu.make_async_copy(
                    sts.at[pl.ds(0, sz)],
                    s_out.at[r, pl.ds(0, sz)], wsem).start()
            return c

        lax.fori_loop(0, BH // 64, row_pair, 0)

        for _ in range(4):
            pltpu.make_async_copy(
                sa0.at[pl.ds(0, sz)],
                a_out.at[0, pl.ds(0, sz)], wsem).wait()


def _make_sc_fill():
    mesh = plsc.VectorSubcoreMesh(core_axis_name="c", subcore_axis_name="s")
    return functools.partial(
        pl.kernel, mesh=mesh,
        out_type=(jax.ShapeDtypeStruct((N, N), jnp.float32),
                  jax.ShapeDtypeStruct((N, N), jnp.float32)),
        scratch_types=[
            pltpu.VMEM((N + 32,), jnp.float32),
            pltpu.VMEM((N + 32,), jnp.float32),
            pltpu.VMEM((N + 32,), jnp.float32),
            pltpu.VMEM((N + 32,), jnp.float32),
            pltpu.VMEM((N + 32,), jnp.float32),
            pltpu.VMEM((N + 32,), jnp.float32),
            pltpu.VMEM((N,), jnp.float32),
            pltpu.VMEM((N,), jnp.float32),
            pltpu.VMEM((N,), jnp.float32),
            pltpu.VMEM((N,), jnp.float32),
            pltpu.SemaphoreType.DMA,
            pltpu.SemaphoreType.DMA,
        ],
    )(_sc_fill_body)


_sc_fill_cache = []


def _sc_fill(lflat):
    if not _sc_fill_cache:
        _sc_fill_cache.append(_make_sc_fill())
    return _sc_fill_cache[0](lflat)


def _merge_body(a_ref, st_ref, l0_ref, l1_ref, out_ref):
    p = pl.program_id(0)
    i = pl.program_id(1)
    j = pl.program_id(2)

    @pl.when(p == 0)
    def _():
        ii = lax.broadcasted_iota(jnp.int32, (BT, BT), 0)
        jj = lax.broadcasted_iota(jnp.int32, (BT, BT), 1)
        eye = (ii == jj).astype(jnp.float32)
        st = lax.dot_general(st_ref[...], eye, (((0,), (0,)), ((), ())),
                             preferred_element_type=jnp.float32)
        a = a_ref[...]
        rows = i * BT + lax.broadcasted_iota(jnp.int32, (BT, BT), 0)
        cols = j * BT + lax.broadcasted_iota(jnp.int32, (BT, BT), 1)
        out_ref[0] = jnp.where(
            cols < rows, a, jnp.where(cols > rows, st, jnp.float32(0.0)))

    @pl.when(p > 0)
    def _():
        out_ref[0] = 1.0 / (1.0 + jnp.exp(l0_ref[0, 0] - l1_ref[0, 0]))


def _zif(p, v):
    return jnp.where(p == 0, v, 0)


_merge = pl.pallas_call(
    _merge_body,
    grid=(3, NT, NT),
    in_specs=[
        pl.BlockSpec((BT, BT), lambda p, i, j: (_zif(p, i), _zif(p, j))),
        pl.BlockSpec((BT, BT), lambda p, i, j: (_zif(p, j), _zif(p, i))),
        pl.BlockSpec(
            (1, 1, BT, BT),
            lambda p, i, j: (0, jnp.maximum(p - 1, 0),
                             jnp.where(p == 0, 0, i), jnp.where(p == 0, 0, j))),
        pl.BlockSpec(
            (1, 1, BT, BT),
            lambda p, i, j: (1, jnp.maximum(p - 1, 0),
                             jnp.where(p == 0, 0, i), jnp.where(p == 0, 0, j))),
    ],
    out_specs=pl.BlockSpec((1, BT, BT), lambda p, i, j: (p, i, j)),
    out_shape=jax.ShapeDtypeStruct((3, N, N), jnp.float32),
)


def kernel(logits, logits_lag):
    a, s = _sc_fill(logits.reshape(-1))
    return _merge(a, s, logits_lag, logits_lag)


# speedup vs baseline: 5.1271x; 5.1271x over previous
import functools

import jax
import jax.numpy as jnp
from jax import lax
from jax.experimental import pallas as pl
from jax.experimental.pallas import tpu as pltpu
from jax.experimental.pallas import tpu_sc as plsc

N = 4096
LAG = 2
N_PAIRS = N * (N - 1) // 2

NW = 32
NBUCK = 8
BH = N // NBUCK

BT = 512
NT = N // BT

CSM = 131072
NSM = (N_PAIRS + CSM - 1) // CSM


def _softmax_body(l_ref, p0_ref, p1_ref):
    e0 = jnp.exp(l_ref[0])
    e1 = jnp.exp(l_ref[1])
    e2 = jnp.exp(l_ref[2])
    inv = 1.0 / (e0 + e1 + e2)
    p0_ref[...] = e0 * inv
    p1_ref[...] = e1 * inv


_softmax = pl.pallas_call(
    _softmax_body,
    grid=(NSM,),
    in_specs=[
        pl.BlockSpec((3, CSM), lambda c: (0, c)),
    ],
    out_specs=[
        pl.BlockSpec((CSM,), lambda c: (c,)),
        pl.BlockSpec((CSM,), lambda c: (c,)),
    ],
    out_shape=[jax.ShapeDtypeStruct((N_PAIRS,), jnp.float32)] * 2,
)


def _sc_scatter_body(p0f, p1f, a_out, s_out,
                     l00, l10, l01, l11,
                     sa0, ss0, sa1, ss1, rsem, wsem):
    wid = lax.axis_index("s") * 2 + lax.axis_index("c")
    lbufs = ((l00, l10), (l01, l11))
    stages = ((sa0, ss0), (sa1, ss1))
    srcs = (p0f, p1f)

    for b in range(NBUCK):
        sz = BH * (b + 1)
        rs = sz + 16
        nv = sz // 16
        clamp_hi = N_PAIRS - rs

        def row_pair(k2, c, b=b, sz=sz, rs=rs, nv=nv, clamp_hi=clamp_hi):
            for half in range(2):
                lb = lbufs[half]
                sta, sts = stages[half]
                k = 2 * k2 + half
                r = BH * b + wid + 32 * k
                off = (r * (r - 1)) // 2
                al = jnp.minimum(off - lax.rem(off, 8), clamp_hi)
                al = pl.multiple_of(al, 8)
                d = off - al

                for p in range(2):
                    pltpu.make_async_copy(
                        srcs[p].at[pl.ds(al, rs)],
                        lb[p].at[pl.ds(0, rs)], rsem).start()

                @pl.when(k2 >= 1)
                def _():
                    pltpu.make_async_copy(
                        sta.at[pl.ds(0, sz)],
                        a_out.at[0, pl.ds(0, sz)], wsem).wait()
                    pltpu.make_async_copy(
                        sts.at[pl.ds(0, sz)],
                        a_out.at[0, pl.ds(0, sz)], wsem).wait()

                for p in range(2):
                    pltpu.make_async_copy(
                        srcs[p].at[pl.ds(al, rs)],
                        lb[p].at[pl.ds(0, rs)], rsem).wait()

                def vec(m, c2, lb=lb, sta=sta, sts=sts, d=d):
                    sta[pl.ds(m * 16, 16)] = lb[0][pl.ds(d + m * 16, 16)]
                    sts[pl.ds(m * 16, 16)] = lb[1][pl.ds(d + m * 16, 16)]
                    return c2

                lax.fori_loop(0, nv, vec, 0)

                pltpu.make_async_copy(
                    sta.at[pl.ds(0, sz)],
                    a_out.at[r, pl.ds(0, sz)], wsem).start()
                pltpu.make_async_copy(
                    sts.at[pl.ds(0, sz)],
                    s_out.at[r, pl.ds(0, sz)], wsem).start()
            return c

        lax.fori_loop(0, BH // 64, row_pair, 0)

        for _ in range(4):
            pltpu.make_async_copy(
                sa0.at[pl.ds(0, sz)],
                a_out.at[0, pl.ds(0, sz)], wsem).wait()


def _make_sc_scatter():
    mesh = plsc.VectorSubcoreMesh(core_axis_name="c", subcore_axis_name="s")
    return functools.partial(
        pl.kernel, mesh=mesh,
        out_type=(jax.ShapeDtypeStruct((N, N), jnp.float32),
                  jax.ShapeDtypeStruct((N, N), jnp.float32)),
        scratch_types=[
            pltpu.VMEM((N + 32,), jnp.float32),
            pltpu.VMEM((N + 32,), jnp.float32),
            pltpu.VMEM((N + 32,), jnp.float32),
            pltpu.VMEM((N + 32,), jnp.float32),
            pltpu.VMEM((N,), jnp.float32),
            pltpu.VMEM((N,), jnp.float32),
            pltpu.VMEM((N,), jnp.float32),
            pltpu.VMEM((N,), jnp.float32),
            pltpu.SemaphoreType.DMA,
            pltpu.SemaphoreType.DMA,
        ],
    )(_sc_scatter_body)


_sc_scatter_cache = []


def _sc_scatter(p0f, p1f):
    if not _sc_scatter_cache:
        _sc_scatter_cache.append(_make_sc_scatter())
    return _sc_scatter_cache[0](p0f, p1f)


def _lag_body(l0_ref, l1_ref, out_ref):
    out_ref[0] = 1.0 / (1.0 + jnp.exp(l0_ref[0, 0] - l1_ref[0, 0]))


_lag = pl.pallas_call(
    _lag_body,
    grid=(LAG, NT, NT),
    in_specs=[
        pl.BlockSpec((1, 1, BT, BT), lambda p, i, j: (0, p, i, j)),
        pl.BlockSpec((1, 1, BT, BT), lambda p, i, j: (1, p, i, j)),
    ],
    out_specs=pl.BlockSpec((1, BT, BT), lambda p, i, j: (p + 1, i, j)),
    out_shape=jax.ShapeDtypeStruct((3, N, N), jnp.float32),
)


def _merge0_body(a_ref, st_ref, base_ref, out_ref):
    i = pl.program_id(0)
    j = pl.program_id(1)
    ii = lax.broadcasted_iota(jnp.int32, (BT, BT), 0)
    jj = lax.broadcasted_iota(jnp.int32, (BT, BT), 1)
    eye = (ii == jj).astype(jnp.float32)
    st = lax.dot_general(st_ref[...], eye, (((0,), (0,)), ((), ())),
                         preferred_element_type=jnp.float32)
    rows = i * BT + ii
    cols = j * BT + jj
    out_ref[0] = jnp.where(
        cols < rows, a_ref[...], jnp.where(cols > rows, st, jnp.float32(0.0)))


_merge0 = pl.pallas_call(
    _merge0_body,
    grid=(NT, NT),
    in_specs=[
        pl.BlockSpec((BT, BT),
                     lambda i, j: (jnp.where(j <= i, i, 0),
                                   jnp.where(j <= i, j, 0))),
        pl.BlockSpec((BT, BT),
                     lambda i, j: (jnp.where(j >= i, j, 0),
                                   jnp.where(j >= i, i, 0))),
        pl.BlockSpec(memory_space=pl.ANY),
    ],
    out_specs=pl.BlockSpec((1, BT, BT), lambda i, j: (0, i, j)),
    out_shape=jax.ShapeDtypeStruct((3, N, N), jnp.float32),
    input_output_aliases={2: 0},
)


def kernel(logits, logits_lag):
    p0f, p1f = _softmax(logits)
    a, s = _sc_scatter(p0f, p1f)
    base = _lag(logits_lag, logits_lag)
    return _merge0(a, s, base)


# speedup vs baseline: 6.7246x; 1.3116x over previous
import functools

import jax
import jax.numpy as jnp
from jax import lax
from jax.experimental import pallas as pl
from jax.experimental.pallas import tpu as pltpu
from jax.experimental.pallas import tpu_sc as plsc

N = 4096
LAG = 2
N_PAIRS = N * (N - 1) // 2

NW = 32
NBUCK = 16
BH = N // NBUCK

BT = 512
NT = N // BT

CSM = 131072
NSM = (N_PAIRS + CSM - 1) // CSM


def _softmax_body(l_ref, p0_ref, p1_ref):
    e0 = jnp.exp(l_ref[0])
    e1 = jnp.exp(l_ref[1])
    e2 = jnp.exp(l_ref[2])
    inv = 1.0 / (e0 + e1 + e2)
    p0_ref[...] = e0 * inv
    p1_ref[...] = e1 * inv


_softmax = pl.pallas_call(
    _softmax_body,
    grid=(NSM,),
    in_specs=[
        pl.BlockSpec((3, CSM), lambda c: (0, c)),
    ],
    out_specs=[
        pl.BlockSpec((CSM,), lambda c: (c,)),
        pl.BlockSpec((CSM,), lambda c: (c,)),
    ],
    out_shape=[jax.ShapeDtypeStruct((N_PAIRS,), jnp.float32)] * 2,
)


NSLOT = 4
RPB = BH // 32


def _row_geom(r, clamp_hi):
    off = (r * (r - 1)) // 2
    al = jnp.minimum(off - lax.rem(off, 8), clamp_hi)
    al = pl.multiple_of(al, 8)
    return al, off - al


def _sc_scatter_body(p0f, p1f, a_out, s_out, *scr):
    lbufs = tuple((scr[2 * s], scr[2 * s + 1]) for s in range(NSLOT))
    stages = tuple((scr[8 + 2 * s], scr[8 + 2 * s + 1]) for s in range(NSLOT))
    rsems = scr[16:20]
    wsems = scr[20:24]
    srcs = (p0f, p1f)
    wid = lax.axis_index("s") * 2 + lax.axis_index("c")

    for b in range(NBUCK):
        sz = BH * (b + 1)
        rs = sz + 16
        nv = sz // 16
        clamp_hi = N_PAIRS - rs
        r0 = BH * b + wid

        def start_reads(k, lb, sem, r0=r0, rs=rs, clamp_hi=clamp_hi):
            al, _ = _row_geom(r0 + 32 * k, clamp_hi)
            for p in range(2):
                pltpu.make_async_copy(
                    srcs[p].at[pl.ds(al, rs)],
                    lb[p].at[pl.ds(0, rs)], sem).start()

        start_reads(0, lbufs[0], rsems[0])
        start_reads(1, lbufs[1], rsems[1])

        def quad(k4, c, b=b, sz=sz, rs=rs, nv=nv, clamp_hi=clamp_hi, r0=r0):
            for half in range(NSLOT):
                lb = lbufs[half]
                sta, sts = stages[half]
                k = NSLOT * k4 + half
                r = r0 + 32 * k
                _, d = _row_geom(r, clamp_hi)

                for p in range(2):
                    pltpu.make_async_copy(
                        srcs[p].at[pl.ds(0, rs)],
                        lb[p].at[pl.ds(0, rs)], rsems[half]).wait()

                @pl.when(k < RPB - 2)
                def _():
                    nslot = (half + 2) % NSLOT
                    start_reads(k + 2, lbufs[nslot], rsems[nslot])

                @pl.when(k >= NSLOT)
                def _():
                    pltpu.make_async_copy(
                        sta.at[pl.ds(0, sz)],
                        a_out.at[0, pl.ds(0, sz)], wsems[half]).wait()
                    pltpu.make_async_copy(
                        sts.at[pl.ds(0, sz)],
                        a_out.at[0, pl.ds(0, sz)], wsems[half]).wait()

                def vec(m, c2, lb=lb, sta=sta, sts=sts, d=d):
                    sta[pl.ds(m * 16, 16)] = lb[0][pl.ds(d + m * 16, 16)]
                    sts[pl.ds(m * 16, 16)] = lb[1][pl.ds(d + m * 16, 16)]
                    return c2

                lax.fori_loop(0, nv, vec, 0)

                pltpu.make_async_copy(
                    sta.at[pl.ds(0, sz)],
                    a_out.at[r, pl.ds(0, sz)], wsems[half]).start()
                pltpu.make_async_copy(
                    sts.at[pl.ds(0, sz)],
                    s_out.at[r, pl.ds(0, sz)], wsems[half]).start()
            return c

        lax.fori_loop(0, RPB // NSLOT, quad, 0)

        for s in range(NSLOT):
            for _ in range(2):
                pltpu.make_async_copy(
                    stages[s][0].at[pl.ds(0, sz)],
                    a_out.at[0, pl.ds(0, sz)], wsems[s]).wait()


def _make_sc_scatter():
    mesh = plsc.VectorSubcoreMesh(core_axis_name="c", subcore_axis_name="s")
    return functools.partial(
        pl.kernel, mesh=mesh,
        out_type=(jax.ShapeDtypeStruct((N, N), jnp.float32),
                  jax.ShapeDtypeStruct((N, N), jnp.float32)),
        scratch_types=(
            [pltpu.VMEM((N + 32,), jnp.float32)] * (2 * NSLOT)
            + [pltpu.VMEM((N,), jnp.float32)] * (2 * NSLOT)
            + [pltpu.SemaphoreType.DMA] * (2 * NSLOT)
        ),
    )(_sc_scatter_body)


_sc_scatter_cache = []


def _sc_scatter(p0f, p1f):
    if not _sc_scatter_cache:
        _sc_scatter_cache.append(_make_sc_scatter())
    return _sc_scatter_cache[0](p0f, p1f)


def _lag_body(l0_ref, l1_ref, out_ref):
    out_ref[0] = 1.0 / (1.0 + jnp.exp(l0_ref[0, 0] - l1_ref[0, 0]))


_lag = pl.pallas_call(
    _lag_body,
    grid=(LAG, NT, NT),
    in_specs=[
        pl.BlockSpec((1, 1, BT, BT), lambda p, i, j: (0, p, i, j)),
        pl.BlockSpec((1, 1, BT, BT), lambda p, i, j: (1, p, i, j)),
    ],
    out_specs=pl.BlockSpec((1, BT, BT), lambda p, i, j: (p + 1, i, j)),
    out_shape=jax.ShapeDtypeStruct((3, N, N), jnp.float32),
)


def _merge0_body(a_ref, st_ref, base_ref, out_ref):
    i = pl.program_id(0)
    j = pl.program_id(1)
    ii = lax.broadcasted_iota(jnp.int32, (BT, BT), 0)
    jj = lax.broadcasted_iota(jnp.int32, (BT, BT), 1)
    eye = (ii == jj).astype(jnp.float32)
    st = lax.dot_general(st_ref[...], eye, (((0,), (0,)), ((), ())),
                         preferred_element_type=jnp.float32)
    rows = i * BT + ii
    cols = j * BT + jj
    out_ref[0] = jnp.where(
        cols < rows, a_ref[...], jnp.where(cols > rows, st, jnp.float32(0.0)))


_merge0 = pl.pallas_call(
    _merge0_body,
    grid=(NT, NT),
    in_specs=[
        pl.BlockSpec((BT, BT),
                     lambda i, j: (jnp.where(j <= i, i, 0),
                                   jnp.where(j <= i, j, 0))),
        pl.BlockSpec((BT, BT),
                     lambda i, j: (jnp.where(j >= i, j, 0),
                                   jnp.where(j >= i, i, 0))),
        pl.BlockSpec(memory_space=pl.ANY),
    ],
    out_specs=pl.BlockSpec((1, BT, BT), lambda i, j: (0, i, j)),
    out_shape=jax.ShapeDtypeStruct((3, N, N), jnp.float32),
    input_output_aliases={2: 0},
)


def kernel(logits, logits_lag):
    p0f, p1f = _softmax(logits)
    a, s = _sc_scatter(p0f, p1f)
    base = _lag(logits_lag, logits_lag)
    return _merge0(a, s, base)


# speedup vs baseline: 6.7943x; 1.0104x over previous
import functools

import jax
import jax.numpy as jnp
from jax import lax
from jax.experimental import pallas as pl
from jax.experimental.pallas import tpu as pltpu
from jax.experimental.pallas import tpu_sc as plsc

N = 4096
LAG = 2
N_PAIRS = N * (N - 1) // 2

NW = 32
NBUCK = 16
BH = N // NBUCK

BT = 512
NT = N // BT

CSM = 131072

R_SPLIT = 2560
I_SPLIT = R_SPLIT // BT
B_SPLIT = R_SPLIT // BH
CHUNKS_A = 25
SZA = CHUNKS_A * CSM
BASE_B = (CHUNKS_A - 1) * CSM
CHUNKS_B = 40
SZB = N_PAIRS - BASE_B


def _softmax_body(l_ref, p0_ref, p1_ref):
    e0 = jnp.exp(l_ref[0])
    e1 = jnp.exp(l_ref[1])
    e2 = jnp.exp(l_ref[2])
    inv = 1.0 / (e0 + e1 + e2)
    p0_ref[...] = e0 * inv
    p1_ref[...] = e1 * inv


def _make_softmax(nchunks, chunk0, out_sz):
    return pl.pallas_call(
        _softmax_body,
        grid=(nchunks,),
        in_specs=[
            pl.BlockSpec((3, CSM), lambda c: (0, c + chunk0)),
        ],
        out_specs=[
            pl.BlockSpec((CSM,), lambda c: (c,)),
            pl.BlockSpec((CSM,), lambda c: (c,)),
        ],
        out_shape=[jax.ShapeDtypeStruct((out_sz,), jnp.float32)] * 2,
    )


_softmax_a = _make_softmax(CHUNKS_A, 0, SZA)
_softmax_b = _make_softmax(CHUNKS_B, CHUNKS_A - 1, SZB)


NSLOT = 4
RPB = BH // 32


def _row_geom(r, clamp_hi, base):
    off = (r * (r - 1)) // 2 - base
    al = jnp.minimum(off - lax.rem(off, 8), clamp_hi)
    al = pl.multiple_of(al, 8)
    return al, off - al


def _sc_scatter_body(p0f, p1f, a_out, s_out, *scr,
                     b_lo=0, b_hi=NBUCK, base=0, src_sz=N_PAIRS):
    lbufs = tuple((scr[2 * s], scr[2 * s + 1]) for s in range(NSLOT))
    stages = tuple((scr[8 + 2 * s], scr[8 + 2 * s + 1]) for s in range(NSLOT))
    rsems = scr[16:20]
    wsems = scr[20:24]
    srcs = (p0f, p1f)
    wid = lax.axis_index("s") * 2 + lax.axis_index("c")

    for b in range(b_lo, b_hi):
        sz = BH * (b + 1)
        rs = sz + 16
        nv = sz // 16
        clamp_hi = src_sz - rs
        r0 = BH * b + wid

        def start_reads(k, lb, sem, r0=r0, rs=rs, clamp_hi=clamp_hi):
            al, _ = _row_geom(r0 + 32 * k, clamp_hi, base)
            for p in range(2):
                pltpu.make_async_copy(
                    srcs[p].at[pl.ds(al, rs)],
                    lb[p].at[pl.ds(0, rs)], sem).start()

        start_reads(0, lbufs[0], rsems[0])
        start_reads(1, lbufs[1], rsems[1])

        def quad(k4, c, b=b, sz=sz, rs=rs, nv=nv, clamp_hi=clamp_hi, r0=r0):
            for half in range(NSLOT):
                lb = lbufs[half]
                sta, sts = stages[half]
                k = NSLOT * k4 + half
                r = r0 + 32 * k
                _, d = _row_geom(r, clamp_hi, base)

                for p in range(2):
                    pltpu.make_async_copy(
                        srcs[p].at[pl.ds(0, rs)],
                        lb[p].at[pl.ds(0, rs)], rsems[half]).wait()

                @pl.when(k < RPB - 2)
                def _():
                    nslot = (half + 2) % NSLOT
                    start_reads(k + 2, lbufs[nslot], rsems[nslot])

                @pl.when(k >= NSLOT)
                def _():
                    pltpu.make_async_copy(
                        sta.at[pl.ds(0, sz)],
                        a_out.at[0, pl.ds(0, sz)], wsems[half]).wait()
                    pltpu.make_async_copy(
                        sts.at[pl.ds(0, sz)],
                        a_out.at[0, pl.ds(0, sz)], wsems[half]).wait()

                def vec(m, c2, lb=lb, sta=sta, sts=sts, d=d):
                    sta[pl.ds(m * 16, 16)] = lb[0][pl.ds(d + m * 16, 16)]
                    sts[pl.ds(m * 16, 16)] = lb[1][pl.ds(d + m * 16, 16)]
                    return c2

                lax.fori_loop(0, nv, vec, 0)

                pltpu.make_async_copy(
                    sta.at[pl.ds(0, sz)],
                    a_out.at[r, pl.ds(0, sz)], wsems[half]).start()
                pltpu.make_async_copy(
                    sts.at[pl.ds(0, sz)],
                    s_out.at[r, pl.ds(0, sz)], wsems[half]).start()
            return c

        lax.fori_loop(0, RPB // NSLOT, quad, 0)

        for s in range(NSLOT):
            for _ in range(2):
                pltpu.make_async_copy(
                    stages[s][0].at[pl.ds(0, sz)],
                    a_out.at[0, pl.ds(0, sz)], wsems[s]).wait()


def _make_sc_scatter(b_lo, b_hi, base, src_sz):
    mesh = plsc.VectorSubcoreMesh(core_axis_name="c", subcore_axis_name="s")
    body = functools.partial(
        _sc_scatter_body, b_lo=b_lo, b_hi=b_hi, base=base, src_sz=src_sz)
    return functools.partial(
        pl.kernel, mesh=mesh,
        out_type=(jax.ShapeDtypeStruct((N, N), jnp.float32),
                  jax.ShapeDtypeStruct((N, N), jnp.float32)),
        scratch_types=(
            [pltpu.VMEM((N + 32,), jnp.float32)] * (2 * NSLOT)
            + [pltpu.VMEM((N,), jnp.float32)] * (2 * NSLOT)
            + [pltpu.SemaphoreType.DMA] * (2 * NSLOT)
        ),
    )(body)


_sc_scatter_cache = []


def _lag_body(l0_ref, l1_ref, out_ref):
    out_ref[0] = 1.0 / (1.0 + jnp.exp(l0_ref[0, 0] - l1_ref[0, 0]))


_lag = pl.pallas_call(
    _lag_body,
    grid=(LAG, NT, NT),
    in_specs=[
        pl.BlockSpec((1, 1, BT, BT), lambda p, i, j: (0, p, i, j)),
        pl.BlockSpec((1, 1, BT, BT), lambda p, i, j: (1, p, i, j)),
    ],
    out_specs=pl.BlockSpec((1, BT, BT), lambda p, i, j: (p + 1, i, j)),
    out_shape=jax.ShapeDtypeStruct((3, N, N), jnp.float32),
)


def _merge0_body(a1_ref, a2_ref, s1_ref, s2_ref, base_ref, out_ref):
    i = pl.program_id(0)
    j = pl.program_id(1)
    ii = lax.broadcasted_iota(jnp.int32, (BT, BT), 0)
    jj = lax.broadcasted_iota(jnp.int32, (BT, BT), 1)
    eye = (ii == jj).astype(jnp.float32)
    a = jnp.where(i < I_SPLIT, a1_ref[...], a2_ref[...])
    s = jnp.where(j < I_SPLIT, s1_ref[...], s2_ref[...])
    st = lax.dot_general(s, eye, (((0,), (0,)), ((), ())),
                         preferred_element_type=jnp.float32)
    rows = i * BT + ii
    cols = j * BT + jj
    out_ref[0] = jnp.where(
        cols < rows, a, jnp.where(cols > rows, st, jnp.float32(0.0)))


def _park(use, i, j):
    return (jnp.where(use, i, 0), jnp.where(use, j, 0))


_merge0 = pl.pallas_call(
    _merge0_body,
    grid=(NT, NT),
    in_specs=[
        pl.BlockSpec((BT, BT),
                     lambda i, j: _park((j <= i) & (i < I_SPLIT), i, j)),
        pl.BlockSpec((BT, BT),
                     lambda i, j: _park((j <= i) & (i >= I_SPLIT), i, j)),
        pl.BlockSpec((BT, BT),
                     lambda i, j: _park((j >= i) & (j < I_SPLIT), j, i)),
        pl.BlockSpec((BT, BT),
                     lambda i, j: _park((j >= i) & (j >= I_SPLIT), j, i)),
        pl.BlockSpec(memory_space=pl.ANY),
    ],
    out_specs=pl.BlockSpec((1, BT, BT), lambda i, j: (0, i, j)),
    out_shape=jax.ShapeDtypeStruct((3, N, N), jnp.float32),
    input_output_aliases={4: 0},
)


def kernel(logits, logits_lag):
    p0a, p1a = _softmax_a(logits)
    if not _sc_scatter_cache:
        _sc_scatter_cache.append(_make_sc_scatter(0, B_SPLIT, 0, SZA))
        _sc_scatter_cache.append(
            _make_sc_scatter(B_SPLIT, NBUCK, BASE_B, SZB))
    a1, s1 = _sc_scatter_cache[0](p0a, p1a)
    p0b, p1b = _softmax_b(logits)
    a2, s2 = _sc_scatter_cache[1](p0b, p1b)
    base = _lag(logits_lag, logits_lag)
    return _merge0(a1, a2, s1, s2, base)


# speedup vs baseline: 7.0470x; 1.0372x over previous
import functools

import jax
import jax.numpy as jnp
from jax import lax
from jax.experimental import pallas as pl
from jax.experimental.pallas import tpu as pltpu
from jax.experimental.pallas import tpu_sc as plsc

N = 4096
LAG = 2
N_PAIRS = N * (N - 1) // 2

NW = 32
NBUCK = 16
BH = N // NBUCK

BT = 512
NT = N // BT

CSM = 262144

R_SPLIT = 2560
I_SPLIT = R_SPLIT // BT
B_SPLIT = R_SPLIT // BH
CHUNKS_A = 13
SZA = CHUNKS_A * CSM
BASE_B = (CHUNKS_A - 1) * CSM
CHUNKS_B = 20
SZB = N_PAIRS - BASE_B


def _softmax_body(l_ref, p0_ref, p1_ref):
    e0 = jnp.exp(l_ref[0])
    e1 = jnp.exp(l_ref[1])
    e2 = jnp.exp(l_ref[2])
    inv = 1.0 / (e0 + e1 + e2)
    p0_ref[...] = e0 * inv
    p1_ref[...] = e1 * inv


def _make_softmax(nchunks, chunk0, out_sz):
    return pl.pallas_call(
        _softmax_body,
        grid=(nchunks,),
        in_specs=[
            pl.BlockSpec((3, CSM), lambda c: (0, c + chunk0)),
        ],
        out_specs=[
            pl.BlockSpec((CSM,), lambda c: (c,)),
            pl.BlockSpec((CSM,), lambda c: (c,)),
        ],
        out_shape=[jax.ShapeDtypeStruct((out_sz,), jnp.float32)] * 2,
    )


_softmax_a = _make_softmax(CHUNKS_A, 0, SZA)
_softmax_b = _make_softmax(CHUNKS_B, CHUNKS_A - 1, SZB)


NSLOT = 4
RPB = BH // 32


def _row_geom(r, clamp_hi, base):
    off = (r * (r - 1)) // 2 - base
    al = jnp.minimum(off - lax.rem(off, 8), clamp_hi)
    al = pl.multiple_of(al, 8)
    return al, off - al


def _sc_scatter_body(p0f, p1f, a_out, s_out, *scr,
                     b_lo=0, b_hi=NBUCK, base=0, src_sz=N_PAIRS):
    lbufs = tuple((scr[2 * s], scr[2 * s + 1]) for s in range(NSLOT))
    stages = tuple((scr[8 + 2 * s], scr[8 + 2 * s + 1]) for s in range(NSLOT))
    rsems = scr[16:20]
    wsems = scr[20:24]
    srcs = (p0f, p1f)
    wid = lax.axis_index("s") * 2 + lax.axis_index("c")

    for b in range(b_lo, b_hi):
        sz = BH * (b + 1)
        rs = sz + 16
        nv = sz // 16
        clamp_hi = src_sz - rs
        r0 = BH * b + wid

        def start_reads(k, lb, sem, r0=r0, rs=rs, clamp_hi=clamp_hi):
            al, _ = _row_geom(r0 + 32 * k, clamp_hi, base)
            for p in range(2):
                pltpu.make_async_copy(
                    srcs[p].at[pl.ds(al, rs)],
                    lb[p].at[pl.ds(0, rs)], sem).start()

        start_reads(0, lbufs[0], rsems[0])
        start_reads(1, lbufs[1], rsems[1])

        def quad(k4, c, b=b, sz=sz, rs=rs, nv=nv, clamp_hi=clamp_hi, r0=r0):
            for half in range(NSLOT):
                lb = lbufs[half]
                sta, sts = stages[half]
                k = NSLOT * k4 + half
                r = r0 + 32 * k
                _, d = _row_geom(r, clamp_hi, base)

                for p in range(2):
                    pltpu.make_async_copy(
                        srcs[p].at[pl.ds(0, rs)],
                        lb[p].at[pl.ds(0, rs)], rsems[half]).wait()

                @pl.when(k < RPB - 2)
                def _():
                    nslot = (half + 2) % NSLOT
                    start_reads(k + 2, lbufs[nslot], rsems[nslot])

                @pl.when(k >= NSLOT)
                def _():
                    pltpu.make_async_copy(
                        sta.at[pl.ds(0, sz)],
                        a_out.at[0, pl.ds(0, sz)], wsems[half]).wait()
                    pltpu.make_async_copy(
                        sts.at[pl.ds(0, sz)],
                        a_out.at[0, pl.ds(0, sz)], wsems[half]).wait()

                def vec(m, c2, lb=lb, sta=sta, sts=sts, d=d):
                    sta[pl.ds(m * 16, 16)] = lb[0][pl.ds(d + m * 16, 16)]
                    sts[pl.ds(m * 16, 16)] = lb[1][pl.ds(d + m * 16, 16)]
                    return c2

                lax.fori_loop(0, nv, vec, 0)

                pltpu.make_async_copy(
                    sta.at[pl.ds(0, sz)],
                    a_out.at[r, pl.ds(0, sz)], wsems[half]).start()
                pltpu.make_async_copy(
                    sts.at[pl.ds(0, sz)],
                    s_out.at[r, pl.ds(0, sz)], wsems[half]).start()
            return c

        lax.fori_loop(0, RPB // NSLOT, quad, 0)

        for s in range(NSLOT):
            for _ in range(2):
                pltpu.make_async_copy(
                    stages[s][0].at[pl.ds(0, sz)],
                    a_out.at[0, pl.ds(0, sz)], wsems[s]).wait()


def _make_sc_scatter(b_lo, b_hi, base, src_sz):
    mesh = plsc.VectorSubcoreMesh(core_axis_name="c", subcore_axis_name="s")
    body = functools.partial(
        _sc_scatter_body, b_lo=b_lo, b_hi=b_hi, base=base, src_sz=src_sz)
    return functools.partial(
        pl.kernel, mesh=mesh,
        out_type=(jax.ShapeDtypeStruct((N, N), jnp.float32),
                  jax.ShapeDtypeStruct((N, N), jnp.float32)),
        scratch_types=(
            [pltpu.VMEM((N + 32,), jnp.float32)] * (2 * NSLOT)
            + [pltpu.VMEM((N,), jnp.float32)] * (2 * NSLOT)
            + [pltpu.SemaphoreType.DMA] * (2 * NSLOT)
        ),
    )(body)


_sc_scatter_cache = []


def _lag_body(l0_ref, l1_ref, out_ref):
    out_ref[0] = 1.0 / (1.0 + jnp.exp(l0_ref[0, 0] - l1_ref[0, 0]))


_lag = pl.pallas_call(
    _lag_body,
    grid=(LAG, NT),
    in_specs=[
        pl.BlockSpec((1, 1, BT, N), lambda p, i: (0, p, i, 0)),
        pl.BlockSpec((1, 1, BT, N), lambda p, i: (1, p, i, 0)),
    ],
    out_specs=pl.BlockSpec((1, BT, N), lambda p, i: (p + 1, i, 0)),
    out_shape=jax.ShapeDtypeStruct((3, N, N), jnp.float32),
)


TSUB = 128


def _mxu_transpose(s):
    nb = BT // TSUB
    ii = lax.broadcasted_iota(jnp.int32, (TSUB, TSUB), 0)
    jj = lax.broadcasted_iota(jnp.int32, (TSUB, TSUB), 1)
    eye = (ii == jj).astype(jnp.float32)
    rows = []
    for u in range(nb):
        row = []
        for v in range(nb):
            blk = s[v * TSUB:(v + 1) * TSUB, u * TSUB:(u + 1) * TSUB]
            row.append(lax.dot_general(blk, eye, (((0,), (0,)), ((), ())),
                                       preferred_element_type=jnp.float32))
        rows.append(jnp.concatenate(row, axis=1))
    return jnp.concatenate(rows, axis=0)


def _merge0_body(a1_ref, a2_ref, s1_ref, s2_ref, base_ref, out_ref):
    i = pl.program_id(0)
    j = pl.program_id(1)

    @pl.when(j < i)
    def _():
        out_ref[0] = jnp.where(i < I_SPLIT, a1_ref[...], a2_ref[...])

    @pl.when(j > i)
    def _():
        s = jnp.where(j < I_SPLIT, s1_ref[...], s2_ref[...])
        out_ref[0] = _mxu_transpose(s)

    @pl.when(j == i)
    def _():
        a = jnp.where(i < I_SPLIT, a1_ref[...], a2_ref[...])
        s = jnp.where(j < I_SPLIT, s1_ref[...], s2_ref[...])
        st = _mxu_transpose(s)
        ii = lax.broadcasted_iota(jnp.int32, (BT, BT), 0)
        jj = lax.broadcasted_iota(jnp.int32, (BT, BT), 1)
        out_ref[0] = jnp.where(
            jj < ii, a, jnp.where(jj > ii, st, jnp.float32(0.0)))


def _park(use, i, j):
    return (jnp.where(use, i, 0), jnp.where(use, j, 0))


_merge0 = pl.pallas_call(
    _merge0_body,
    grid=(NT, NT),
    in_specs=[
        pl.BlockSpec((BT, BT),
                     lambda i, j: _park((j <= i) & (i < I_SPLIT), i, j)),
        pl.BlockSpec((BT, BT),
                     lambda i, j: _park((j <= i) & (i >= I_SPLIT), i, j)),
        pl.BlockSpec((BT, BT),
                     lambda i, j: _park((j >= i) & (j < I_SPLIT), j, i)),
        pl.BlockSpec((BT, BT),
                     lambda i, j: _park((j >= i) & (j >= I_SPLIT), j, i)),
        pl.BlockSpec(memory_space=pl.ANY),
    ],
    out_specs=pl.BlockSpec((1, BT, BT), lambda i, j: (0, i, j)),
    out_shape=jax.ShapeDtypeStruct((3, N, N), jnp.float32),
    input_output_aliases={4: 0},
)


def kernel(logits, logits_lag):
    p0a, p1a = _softmax_a(logits)
    if not _sc_scatter_cache:
        _sc_scatter_cache.append(_make_sc_scatter(0, B_SPLIT, 0, SZA))
        _sc_scatter_cache.append(
            _make_sc_scatter(B_SPLIT, NBUCK, BASE_B, SZB))
    a1, s1 = _sc_scatter_cache[0](p0a, p1a)
    p0b, p1b = _softmax_b(logits)
    a2, s2 = _sc_scatter_cache[1](p0b, p1b)
    base = _lag(logits_lag, logits_lag)
    return _merge0(a1, a2, s1, s2, base)
